# Initial kernel scaffold; baseline (speedup 1.0000x reference)
import jax
import jax.numpy as jnp
from jax.experimental import pallas as pl


def kernel(x, pos, edge_index, batch, emb_W1, emb_b1, emb_W2, emb_b2, msg1_W, msg1_b, msg2_W, msg2_b, upd1_W, upd1_b, upd2_W, upd2_b, dbl_W, dbl_b, conv1_W, conv1_b, conv2_W, conv2_b):
    raise NotImplementedError("write your pallas kernel here")


# speedup vs baseline: 3.3008x; 3.3008x over previous
import functools

import jax
import jax.numpy as jnp
import numpy as np
from jax import lax
from jax.experimental import pallas as pl
from jax.experimental.pallas import tpu as pltpu
from jax.experimental.pallas import tpu_sc as plsc

N = 10000
E = 160000
TW = 25
H = 128
NLAYERS = 6
PDE_L = 16.0
PDE_TMAX = 4.0
PDE_DT = 0.01

_INTERPRET = False

NP = 10240
EP = 163840
CHUNK = 128
NCHUNKS = EP // CHUNK
NWORK = 32
CPW = NCHUNKS // NWORK
NPS = NP // 16
NBUF = 4
NGRP = CPW // NBUF
NBUF_S = 2
NGRP_S = CPW // NBUF_S


def _swish(v):
    return v * jax.nn.sigmoid(v)


def _enc_body(F_ref, deg2_ref, eW1_ref, eb1_ref, eW2_ref, eb2_ref,
              WPh_ref, WPf_ref, WQh_ref, WQf_ref, b1_ref,
              h_ref, P_ref, Q_ref, degi_ref):
    F = F_ref[...]
    h = _swish(jnp.dot(F, eW1_ref[...], preferred_element_type=jnp.float32)
               + eb1_ref[...])
    h = _swish(jnp.dot(h, eW2_ref[...], preferred_element_type=jnp.float32)
               + eb2_ref[...])
    h_ref[...] = h
    P_ref[...] = (jnp.dot(h, WPh_ref[...], preferred_element_type=jnp.float32)
                  + jnp.dot(F, WPf_ref[...], preferred_element_type=jnp.float32)
                  + b1_ref[...])
    Q_ref[...] = (jnp.dot(h, WQh_ref[...], preferred_element_type=jnp.float32)
                  + jnp.dot(F, WQf_ref[...], preferred_element_type=jnp.float32))
    deg = deg2_ref[0, :, 0:8] + deg2_ref[1, :, 0:8]
    degi_ref[...] = 1.0 / jnp.maximum(deg, 1.0)


def _enc_call(F, deg2, eW1, eb1, eW2, eb2, WPh, WPf, WQh, WQf, b1):
    return pl.pallas_call(
        _enc_body,
        out_shape=(
            jax.ShapeDtypeStruct((NP, H), jnp.float32),
            jax.ShapeDtypeStruct((NP, H), jnp.float32),
            jax.ShapeDtypeStruct((NP, H), jnp.float32),
            jax.ShapeDtypeStruct((NP, 8), jnp.float32),
        ),
        interpret=_INTERPRET,
    )(F, deg2, eW1, eb1, eW2, eb2, WPh, WPf, WQh, WQf, b1)


def _msg_body(t_ref, W2_ref, b2_ref, m2_ref):
    m = _swish(t_ref[...])
    m2_ref[...] = _swish(
        jnp.dot(m, W2_ref[...], preferred_element_type=jnp.float32)
        + b2_ref[...])


def _msg_call(t, W2, b2):
    blk = 2048
    grid = EP // blk
    return pl.pallas_call(
        _msg_body,
        grid=(grid,),
        in_specs=[
            pl.BlockSpec((blk, H), lambda i: (i, 0)),
            pl.BlockSpec((H, H), lambda i: (0, 0)),
            pl.BlockSpec((H,), lambda i: (0,)),
        ],
        out_specs=pl.BlockSpec((blk, H), lambda i: (i, 0)),
        out_shape=jax.ShapeDtypeStruct((EP, H), jnp.float32),
        interpret=_INTERPRET,
    )(t, W2, b2)


def _upd_common(h_ref, F_ref, agg2_ref, degi_ref,
                Uh_ref, Ua_ref, Uv_ref, ub1_ref, U2_ref, ub2_ref):
    h = h_ref[...]
    agg = (agg2_ref[0] + agg2_ref[1]) * degi_ref[:, 0:1]
    pt = F_ref[:, 51:52]
    u1 = _swish(jnp.dot(h, Uh_ref[...], preferred_element_type=jnp.float32)
                + jnp.dot(agg, Ua_ref[...], preferred_element_type=jnp.float32)
                + pt * Uv_ref[...]
                + ub1_ref[...])
    u2 = _swish(jnp.dot(u1, U2_ref[...], preferred_element_type=jnp.float32)
                + ub2_ref[...])
    hn = h + u2
    mean = jnp.mean(hn[:N], axis=0, keepdims=True)
    hc = hn - mean
    var = jnp.mean(hc[:N] * hc[:N], axis=0, keepdims=True)
    return hc * jax.lax.rsqrt(var + 1e-5)


def _upd_body(h_ref, F_ref, agg2_ref, degi_ref,
              Uh_ref, Ua_ref, Uv_ref, ub1_ref, U2_ref, ub2_ref,
              WPh_ref, WPf_ref, WQh_ref, WQf_ref, b1_ref,
              h_out, P_ref, Q_ref):
    hn = _upd_common(h_ref, F_ref, agg2_ref, degi_ref,
                     Uh_ref, Ua_ref, Uv_ref, ub1_ref, U2_ref, ub2_ref)
    F = F_ref[...]
    h_out[...] = hn
    P_ref[...] = (jnp.dot(hn, WPh_ref[...], preferred_element_type=jnp.float32)
                  + jnp.dot(F, WPf_ref[...], preferred_element_type=jnp.float32)
                  + b1_ref[...])
    Q_ref[...] = (jnp.dot(hn, WQh_ref[...], preferred_element_type=jnp.float32)
                  + jnp.dot(F, WQf_ref[...], preferred_element_type=jnp.float32))


def _upd_call(h, F, agg2, degi, Uh, Ua, Uv, ub1, U2, ub2,
              WPh, WPf, WQh, WQf, b1):
    return pl.pallas_call(
        _upd_body,
        out_shape=(
            jax.ShapeDtypeStruct((NP, H), jnp.float32),
            jax.ShapeDtypeStruct((NP, H), jnp.float32),
            jax.ShapeDtypeStruct((NP, H), jnp.float32),
        ),
        interpret=_INTERPRET,
    )(h, F, agg2, degi, Uh, Ua, Uv, ub1, U2, ub2, WPh, WPf, WQh, WQf, b1)


def _fin_body(h_ref, F_ref, agg2_ref, degi_ref,
              Uh_ref, Ua_ref, Uv_ref, ub1_ref, U2_ref, ub2_ref,
              dblW_ref, dblb_ref, K1_ref, b1r_ref, K2_ref, b2r_ref,
              upad_ref, dtv_ref, out_ref):
    hn = _upd_common(h_ref, F_ref, agg2_ref, degi_ref,
                     Uh_ref, Ua_ref, Uv_ref, ub1_ref, U2_ref, ub2_ref)
    h2 = _swish(jnp.dot(hn, dblW_ref[...], preferred_element_type=jnp.float32)
                + dblb_ref[...])
    c = _swish(jnp.dot(h2, K1_ref[...], preferred_element_type=jnp.float32)
               + b1r_ref[...])
    diff = (jnp.dot(c, K2_ref[...], preferred_element_type=jnp.float32)
            + b2r_ref[...])
    out_ref[...] = upad_ref[...] + dtv_ref[...] * diff


def _fin_call(h, F, agg2, degi, Uh, Ua, Uv, ub1, U2, ub2,
              dblW, dblb, K1, b1r, K2, b2r, upad, dtv):
    return pl.pallas_call(
        _fin_body,
        out_shape=jax.ShapeDtypeStruct((NP, 64), jnp.float32),
        interpret=_INTERPRET,
    )(h, F, agg2, degi, Uh, Ua, Uv, ub1, U2, ub2,
      dblW, dblb, K1, b1r, K2, b2r, upad, dtv)


_MESH = plsc.VectorSubcoreMesh(core_axis_name="c", subcore_axis_name="s")


def _wid():
    return lax.axis_index("s") * 2 + lax.axis_index("c")


@functools.partial(
    pl.kernel,
    out_type=jax.ShapeDtypeStruct((EP, H), jnp.float32),
    mesh=_MESH,
    scratch_types=[
        pltpu.VMEM((CPW, CHUNK), jnp.int32),
        pltpu.VMEM((CPW, CHUNK), jnp.int32),
        pltpu.VMEM((NBUF, CHUNK, H), jnp.float32),
        pltpu.SemaphoreType.DMA,
        pltpu.SemaphoreType.DMA,
        pltpu.SemaphoreType.DMA,
        pltpu.SemaphoreType.DMA,
    ],
)
def _sc_gather_kernel(P_hbm, Q_hbm, dst2_hbm, src2_hbm, t_hbm,
                      idxd_v, idxs_v, rows_v, s0, s1, s2, s3):
    w = _wid()
    sems = (s0, s1, s2, s3)
    pltpu.sync_copy(dst2_hbm.at[pl.ds(w * CPW, CPW)], idxd_v)
    pltpu.sync_copy(src2_hbm.at[pl.ds(w * CPW, CPW)], idxs_v)

    def group(g, _):
        cps = [pltpu.async_copy(P_hbm.at[idxd_v.at[g * NBUF + b]],
                                rows_v.at[b], sems[b])
               for b in range(NBUF)]
        cqs = []
        for b in range(NBUF):
            cps[b].wait()
            cqs.append(pltpu.async_copy(Q_hbm.at[idxs_v.at[g * NBUF + b]],
                                        rows_v.at[b], sems[b], add=True))
        cws = []
        for b in range(NBUF):
            cqs[b].wait()
            row = w * CPW + g * NBUF + b
            cws.append(pltpu.async_copy(rows_v.at[b],
                                        t_hbm.at[pl.ds(row * CHUNK, CHUNK)],
                                        sems[b]))
        for b in range(NBUF):
            cws[b].wait()
        return 0

    lax.fori_loop(0, NGRP, group, 0)


def _sc_gather(P, Q, dst2, src2):
    return _sc_gather_kernel(P, Q, dst2, src2)


@functools.partial(
    pl.kernel,
    out_type=jax.ShapeDtypeStruct((2, NP, H), jnp.float32),
    mesh=_MESH,
    scratch_types=[
        pltpu.VMEM((CPW, CHUNK), jnp.int32),
        pltpu.VMEM((NBUF_S, CHUNK, H), jnp.float32),
        pltpu.VMEM_SHARED((NP, H), jnp.float32),
        pltpu.SemaphoreType.DMA,
        pltpu.SemaphoreType.DMA,
    ],
)
def _sc_scatter_kernel(m2_hbm, dst2_hbm, z_hbm, agg2_hbm,
                       idx_v, buf_v, agg_sh, s0, s1):
    c = lax.axis_index("c")
    s = lax.axis_index("s")
    sems = (s0, s1)
    pltpu.sync_copy(z_hbm, agg_sh.at[pl.ds(s * NPS, NPS)])
    plsc.subcore_barrier()
    w = _wid()
    pltpu.sync_copy(dst2_hbm.at[pl.ds(w * CPW, CPW)], idx_v)

    def group(g, _):
        crs = [pltpu.async_copy(
                   m2_hbm.at[pl.ds((w * CPW + g * NBUF_S + b) * CHUNK, CHUNK)],
                   buf_v.at[b], sems[b])
               for b in range(NBUF_S)]
        css = []
        for b in range(NBUF_S):
            crs[b].wait()
            css.append(pltpu.async_copy(buf_v.at[b],
                                        agg_sh.at[idx_v.at[g * NBUF_S + b]],
                                        sems[b], add=True))
        for b in range(NBUF_S):
            css[b].wait()
        return 0

    lax.fori_loop(0, NGRP_S, group, 0)
    plsc.subcore_barrier()
    pltpu.sync_copy(agg_sh.at[pl.ds(s * NPS, NPS)],
                    agg2_hbm.at[c, pl.ds(s * NPS, NPS)])


def _sc_scatter(m2, dst2, z):
    return _sc_scatter_kernel(m2, dst2, z)


@functools.partial(
    pl.kernel,
    out_type=jax.ShapeDtypeStruct((2, NP, H), jnp.float32),
    mesh=_MESH,
    scratch_types=[
        pltpu.VMEM((CPW, CHUNK), jnp.int32),
        pltpu.VMEM((CHUNK, H), jnp.float32),
        pltpu.VMEM_SHARED((NP, H), jnp.float32),
    ],
)
def _sc_deg_kernel(dst2_hbm, ones_hbm, z16_hbm, deg2_hbm,
                   idx_v, ones_v, deg_sh):
    c = lax.axis_index("c")
    s = lax.axis_index("s")
    pltpu.sync_copy(z16_hbm, deg_sh.at[pl.ds(s * NPS, NPS)])
    pltpu.sync_copy(ones_hbm, ones_v)
    plsc.subcore_barrier()
    w = _wid()
    pltpu.sync_copy(dst2_hbm.at[pl.ds(w * CPW, CPW)], idx_v)

    def body(i, _):
        pltpu.sync_copy(ones_v, deg_sh.at[idx_v.at[i]], add=True)
        return 0

    lax.fori_loop(0, CPW, body, 0)
    plsc.subcore_barrier()
    pltpu.sync_copy(deg_sh.at[pl.ds(s * NPS, NPS)],
                    deg2_hbm.at[c, pl.ds(s * NPS, NPS)])


def _sc_deg(dst2):
    ones = jnp.ones((CHUNK, H), jnp.float32)
    z16 = jnp.zeros((NPS, H), jnp.float32)
    return _sc_deg_kernel(dst2, ones, z16)


def kernel(x, pos, edge_index, batch, emb_W1, emb_b1, emb_W2, emb_b2,
           msg1_W, msg1_b, msg2_W, msg2_b, upd1_W, upd1_b, upd2_W, upd2_b,
           dbl_W, dbl_b, conv1_W, conv1_b, conv2_W, conv2_b):
    u = x
    pos_x = pos[:, 1:2] / PDE_L
    pos_t = pos[:, 0:1] / PDE_TMAX
    F = jnp.pad(jnp.concatenate([u, pos_x, pos_t], axis=1),
                ((0, NP - N), (0, 0)))
    src2 = jnp.pad(edge_index[0], (0, EP - E),
                   constant_values=N).reshape(NCHUNKS, CHUNK)
    dst2 = jnp.pad(edge_index[1], (0, EP - E),
                   constant_values=N).reshape(NCHUNKS, CHUNK)
    zN = jnp.zeros((NPS, H), jnp.float32)

    WPh = msg1_W[:, 0:H, :]
    WQh = msg1_W[:, H:2 * H, :]
    WPf = msg1_W[:, 2 * H:2 * H + 52, :]
    WQf = jnp.concatenate(
        [-msg1_W[:, 2 * H:2 * H + 51, :],
         jnp.zeros((NLAYERS, 1, H), jnp.float32)], axis=1)
    Uh = upd1_W[:, 0:H, :]
    Ua = upd1_W[:, H:2 * H, :]
    Uv = upd1_W[:, 2 * H:2 * H + 1, :]

    o_, i_, k_, j_ = np.meshgrid(np.arange(8), np.arange(2), np.arange(16),
                                 np.arange(38), indexing="ij")
    K1 = jnp.zeros((2 * H, 320), jnp.float32).at[
        (i_ * H + 3 * j_ + k_).ravel(), (o_ * 38 + j_).ravel()
    ].set(conv1_W[o_.ravel(), i_.ravel(), k_.ravel()])
    b1r = jnp.zeros((320,), jnp.float32).at[
        (o_ * 38 + j_).ravel()].set(conv1_b[o_.ravel()])
    c_, o2_, j2_, jj_ = np.meshgrid(np.arange(2), np.arange(8),
                                    np.arange(14), np.arange(25),
                                    indexing="ij")
    K2 = jnp.zeros((320, 64), jnp.float32).at[
        (o2_ * 38 + jj_ + j2_).ravel(), (c_ * 25 + jj_).ravel()
    ].add(conv2_W[c_.ravel(), o2_.ravel(), j2_.ravel()])
    b2r = jnp.zeros((64,), jnp.float32).at[
        (c_ * 25 + jj_).ravel()].set(conv2_b[c_.ravel()])
    dtv = jnp.zeros((64,), jnp.float32).at[np.arange(50)].set(
        np.tile(PDE_DT * (np.arange(25) + 1.0), 2).astype(np.float32))
    upad = jnp.pad(u, ((0, NP - N), (0, 64 - 2 * TW)))

    deg2 = _sc_deg(dst2)
    h, P, Q, degi = _enc_call(F, deg2, emb_W1, emb_b1, emb_W2, emb_b2,
                              WPh[0], WPf[0], WQh[0], WQf[0], msg1_b[0])
    for l in range(NLAYERS):
        t = _sc_gather(P, Q, dst2, src2)
        m2 = _msg_call(t, msg2_W[l], msg2_b[l])
        agg2 = _sc_scatter(m2, dst2, zN)
        if l < NLAYERS - 1:
            h, P, Q = _upd_call(h, F, agg2, degi,
                                Uh[l], Ua[l], Uv[l], upd1_b[l],
                                upd2_W[l], upd2_b[l],
                                WPh[l + 1], WPf[l + 1], WQh[l + 1],
                                WQf[l + 1], msg1_b[l + 1])
        else:
            outp = _fin_call(h, F, agg2, degi,
                             Uh[l], Ua[l], Uv[l], upd1_b[l],
                             upd2_W[l], upd2_b[l],
                             dbl_W, dbl_b, K1, b1r, K2, b2r, upad, dtv)
    return outp[:N, :2 * TW]


# speedup vs baseline: 3.4681x; 1.0507x over previous
import functools

import jax
import jax.numpy as jnp
import numpy as np
from jax import lax
from jax.experimental import pallas as pl
from jax.experimental.pallas import tpu as pltpu
from jax.experimental.pallas import tpu_sc as plsc

N = 10000
E = 160000
TW = 25
H = 128
NLAYERS = 6
PDE_L = 16.0
PDE_TMAX = 4.0
PDE_DT = 0.01

_INTERPRET = False

NP = 10240
EP = 163840
CHUNK = 128
NCHUNKS = EP // CHUNK
NWORK = 32
CPW = NCHUNKS // NWORK
NPS = NP // 16
NBUF = 5
NGRP = CPW // NBUF
NBUF_S = 2
NGRP_S = CPW // NBUF_S


def _swish(v):
    return v * jax.nn.sigmoid(v)


def _enc_body(F_ref, deg2_ref, eW1_ref, eb1_ref, eW2_ref, eb2_ref,
              WPh_ref, WPf_ref, WQh_ref, WQf_ref, b1_ref,
              h_ref, P_ref, Q_ref, degi_ref):
    F = F_ref[...]
    h = _swish(jnp.dot(F, eW1_ref[...], preferred_element_type=jnp.float32)
               + eb1_ref[...])
    h = _swish(jnp.dot(h, eW2_ref[...], preferred_element_type=jnp.float32)
               + eb2_ref[...])
    h_ref[...] = h
    P_ref[...] = (jnp.dot(h, WPh_ref[...], preferred_element_type=jnp.float32)
                  + jnp.dot(F, WPf_ref[...], preferred_element_type=jnp.float32)
                  + b1_ref[...])
    Q_ref[...] = (jnp.dot(h, WQh_ref[...], preferred_element_type=jnp.float32)
                  + jnp.dot(F, WQf_ref[...], preferred_element_type=jnp.float32))
    deg = deg2_ref[0, :, 0:8] + deg2_ref[1, :, 0:8]
    degi_ref[...] = 1.0 / jnp.maximum(deg, 1.0)


def _enc_call(F, deg2, eW1, eb1, eW2, eb2, WPh, WPf, WQh, WQf, b1):
    return pl.pallas_call(
        _enc_body,
        out_shape=(
            jax.ShapeDtypeStruct((NP, H), jnp.float32),
            jax.ShapeDtypeStruct((NP, H), jnp.float32),
            jax.ShapeDtypeStruct((NP, H), jnp.float32),
            jax.ShapeDtypeStruct((NP, 8), jnp.float32),
        ),
        interpret=_INTERPRET,
    )(F, deg2, eW1, eb1, eW2, eb2, WPh, WPf, WQh, WQf, b1)


def _msg_body(t_ref, W2_ref, b2_ref, m2_ref):
    m = _swish(t_ref[...])
    m2_ref[...] = _swish(
        jnp.dot(m, W2_ref[...], preferred_element_type=jnp.float32)
        + b2_ref[...])


def _msg_call(t, W2, b2):
    blk = 2048
    grid = EP // blk
    return pl.pallas_call(
        _msg_body,
        grid=(grid,),
        in_specs=[
            pl.BlockSpec((blk, H), lambda i: (i, 0)),
            pl.BlockSpec((H, H), lambda i: (0, 0)),
            pl.BlockSpec((H,), lambda i: (0,)),
        ],
        out_specs=pl.BlockSpec((blk, H), lambda i: (i, 0)),
        out_shape=jax.ShapeDtypeStruct((EP, H), jnp.float32),
        interpret=_INTERPRET,
    )(t, W2, b2)


def _upd_common(h_ref, F_ref, agg2_ref, degi_ref,
                Uh_ref, Ua_ref, Uv_ref, ub1_ref, U2_ref, ub2_ref):
    h = h_ref[...]
    agg = (agg2_ref[0] + agg2_ref[1]) * degi_ref[:, 0:1]
    pt = F_ref[:, 51:52]
    u1 = _swish(jnp.dot(h, Uh_ref[...], preferred_element_type=jnp.float32)
                + jnp.dot(agg, Ua_ref[...], preferred_element_type=jnp.float32)
                + pt * Uv_ref[...]
                + ub1_ref[...])
    u2 = _swish(jnp.dot(u1, U2_ref[...], preferred_element_type=jnp.float32)
                + ub2_ref[...])
    hn = h + u2
    mean = jnp.mean(hn[:N], axis=0, keepdims=True)
    hc = hn - mean
    var = jnp.mean(hc[:N] * hc[:N], axis=0, keepdims=True)
    return hc * jax.lax.rsqrt(var + 1e-5)


def _upd_body(h_ref, F_ref, agg2_ref, degi_ref,
              Uh_ref, Ua_ref, Uv_ref, ub1_ref, U2_ref, ub2_ref,
              WPh_ref, WPf_ref, WQh_ref, WQf_ref, b1_ref,
              h_out, P_ref, Q_ref):
    hn = _upd_common(h_ref, F_ref, agg2_ref, degi_ref,
                     Uh_ref, Ua_ref, Uv_ref, ub1_ref, U2_ref, ub2_ref)
    F = F_ref[...]
    h_out[...] = hn
    P_ref[...] = (jnp.dot(hn, WPh_ref[...], preferred_element_type=jnp.float32)
                  + jnp.dot(F, WPf_ref[...], preferred_element_type=jnp.float32)
                  + b1_ref[...])
    Q_ref[...] = (jnp.dot(hn, WQh_ref[...], preferred_element_type=jnp.float32)
                  + jnp.dot(F, WQf_ref[...], preferred_element_type=jnp.float32))


def _upd_call(h, F, agg2, degi, Uh, Ua, Uv, ub1, U2, ub2,
              WPh, WPf, WQh, WQf, b1):
    return pl.pallas_call(
        _upd_body,
        out_shape=(
            jax.ShapeDtypeStruct((NP, H), jnp.float32),
            jax.ShapeDtypeStruct((NP, H), jnp.float32),
            jax.ShapeDtypeStruct((NP, H), jnp.float32),
        ),
        interpret=_INTERPRET,
    )(h, F, agg2, degi, Uh, Ua, Uv, ub1, U2, ub2, WPh, WPf, WQh, WQf, b1)


def _fin_body(h_ref, F_ref, agg2_ref, degi_ref,
              Uh_ref, Ua_ref, Uv_ref, ub1_ref, U2_ref, ub2_ref,
              dblW_ref, dblb_ref, K1_ref, b1r_ref, K2_ref, b2r_ref,
              upad_ref, dtv_ref, out_ref):
    hn = _upd_common(h_ref, F_ref, agg2_ref, degi_ref,
                     Uh_ref, Ua_ref, Uv_ref, ub1_ref, U2_ref, ub2_ref)
    h2 = _swish(jnp.dot(hn, dblW_ref[...], preferred_element_type=jnp.float32)
                + dblb_ref[...])
    c = _swish(jnp.dot(h2, K1_ref[...], preferred_element_type=jnp.float32)
               + b1r_ref[...])
    diff = (jnp.dot(c, K2_ref[...], preferred_element_type=jnp.float32)
            + b2r_ref[...])
    out_ref[...] = upad_ref[...] + dtv_ref[...] * diff


def _fin_call(h, F, agg2, degi, Uh, Ua, Uv, ub1, U2, ub2,
              dblW, dblb, K1, b1r, K2, b2r, upad, dtv):
    return pl.pallas_call(
        _fin_body,
        out_shape=jax.ShapeDtypeStruct((NP, 64), jnp.float32),
        interpret=_INTERPRET,
    )(h, F, agg2, degi, Uh, Ua, Uv, ub1, U2, ub2,
      dblW, dblb, K1, b1r, K2, b2r, upad, dtv)


_MESH = plsc.VectorSubcoreMesh(core_axis_name="c", subcore_axis_name="s")


def _wid():
    return lax.axis_index("s") * 2 + lax.axis_index("c")


@functools.partial(
    pl.kernel,
    out_type=jax.ShapeDtypeStruct((EP, H), jnp.float32),
    mesh=_MESH,
    scratch_types=[
        pltpu.VMEM((CPW, CHUNK), jnp.int32),
        pltpu.VMEM((CPW, CHUNK), jnp.int32),
        pltpu.VMEM((NBUF, CHUNK, H), jnp.float32),
        pltpu.SemaphoreType.DMA,
        pltpu.SemaphoreType.DMA,
        pltpu.SemaphoreType.DMA,
        pltpu.SemaphoreType.DMA,
        pltpu.SemaphoreType.DMA,
    ],
)
def _sc_gather_kernel(P_hbm, Q_hbm, dst2_hbm, src2_hbm, t_hbm,
                      idxd_v, idxs_v, rows_v, s0, s1, s2, s3, s4):
    w = _wid()
    sems = (s0, s1, s2, s3, s4)
    pltpu.sync_copy(dst2_hbm.at[pl.ds(w * CPW, CPW)], idxd_v)
    pltpu.sync_copy(src2_hbm.at[pl.ds(w * CPW, CPW)], idxs_v)

    def group(g, _):
        cps = [pltpu.async_copy(P_hbm.at[idxd_v.at[g * NBUF + b]],
                                rows_v.at[b], sems[b])
               for b in range(NBUF)]
        cqs = []
        for b in range(NBUF):
            cps[b].wait()
            cqs.append(pltpu.async_copy(Q_hbm.at[idxs_v.at[g * NBUF + b]],
                                        rows_v.at[b], sems[b], add=True))
        cws = []
        for b in range(NBUF):
            cqs[b].wait()
            row = w * CPW + g * NBUF + b
            cws.append(pltpu.async_copy(rows_v.at[b],
                                        t_hbm.at[pl.ds(row * CHUNK, CHUNK)],
                                        sems[b]))
        for b in range(NBUF):
            cws[b].wait()
        return 0

    lax.fori_loop(0, NGRP, group, 0)


def _sc_gather(P, Q, dst2, src2):
    return _sc_gather_kernel(P, Q, dst2, src2)


@functools.partial(
    pl.kernel,
    out_type=jax.ShapeDtypeStruct((2, NP, H), jnp.float32),
    mesh=_MESH,
    scratch_types=[
        pltpu.VMEM((CPW, CHUNK), jnp.int32),
        pltpu.VMEM((NBUF_S, CHUNK, H), jnp.float32),
        pltpu.VMEM_SHARED((NP, H), jnp.float32),
        pltpu.SemaphoreType.DMA,
        pltpu.SemaphoreType.DMA,
    ],
)
def _sc_scatter_kernel(m2_hbm, dst2_hbm, z_hbm, agg2_hbm,
                       idx_v, buf_v, agg_sh, s0, s1):
    c = lax.axis_index("c")
    s = lax.axis_index("s")
    sems = (s0, s1)
    pltpu.sync_copy(z_hbm, agg_sh.at[pl.ds(s * NPS, NPS)])
    plsc.subcore_barrier()
    w = _wid()
    pltpu.sync_copy(dst2_hbm.at[pl.ds(w * CPW, CPW)], idx_v)

    def group(g, _):
        crs = [pltpu.async_copy(
                   m2_hbm.at[pl.ds((w * CPW + g * NBUF_S + b) * CHUNK, CHUNK)],
                   buf_v.at[b], sems[b])
               for b in range(NBUF_S)]
        css = []
        for b in range(NBUF_S):
            crs[b].wait()
            css.append(pltpu.async_copy(buf_v.at[b],
                                        agg_sh.at[idx_v.at[g * NBUF_S + b]],
                                        sems[b], add=True))
        for b in range(NBUF_S):
            css[b].wait()
        return 0

    lax.fori_loop(0, NGRP_S, group, 0)
    plsc.subcore_barrier()
    pltpu.sync_copy(agg_sh.at[pl.ds(s * NPS, NPS)],
                    agg2_hbm.at[c, pl.ds(s * NPS, NPS)])


def _sc_scatter(m2, dst2, z):
    return _sc_scatter_kernel(m2, dst2, z)


@functools.partial(
    pl.kernel,
    out_type=jax.ShapeDtypeStruct((2, NP, H), jnp.float32),
    mesh=_MESH,
    scratch_types=[
        pltpu.VMEM((CPW, CHUNK), jnp.int32),
        pltpu.VMEM((CHUNK, H), jnp.float32),
        pltpu.VMEM_SHARED((NP, H), jnp.float32),
        pltpu.SemaphoreType.DMA,
        pltpu.SemaphoreType.DMA,
        pltpu.SemaphoreType.DMA,
        pltpu.SemaphoreType.DMA,
    ],
)
def _sc_deg_kernel(dst2_hbm, ones_hbm, z16_hbm, deg2_hbm,
                   idx_v, ones_v, deg_sh, s0, s1, s2, s3):
    c = lax.axis_index("c")
    s = lax.axis_index("s")
    sems = (s0, s1, s2, s3)
    pltpu.sync_copy(z16_hbm, deg_sh.at[pl.ds(s * NPS, NPS)])
    pltpu.sync_copy(ones_hbm, ones_v)
    plsc.subcore_barrier()
    w = _wid()
    pltpu.sync_copy(dst2_hbm.at[pl.ds(w * CPW, CPW)], idx_v)

    def group(g, _):
        cs = [pltpu.async_copy(ones_v, deg_sh.at[idx_v.at[g * 4 + b]],
                               sems[b], add=True)
              for b in range(4)]
        for b in range(4):
            cs[b].wait()
        return 0

    lax.fori_loop(0, CPW // 4, group, 0)
    plsc.subcore_barrier()
    pltpu.sync_copy(deg_sh.at[pl.ds(s * NPS, NPS)],
                    deg2_hbm.at[c, pl.ds(s * NPS, NPS)])


def _sc_deg(dst2):
    ones = jnp.ones((CHUNK, H), jnp.float32)
    z16 = jnp.zeros((NPS, H), jnp.float32)
    return _sc_deg_kernel(dst2, ones, z16)


def kernel(x, pos, edge_index, batch, emb_W1, emb_b1, emb_W2, emb_b2,
           msg1_W, msg1_b, msg2_W, msg2_b, upd1_W, upd1_b, upd2_W, upd2_b,
           dbl_W, dbl_b, conv1_W, conv1_b, conv2_W, conv2_b):
    u = x
    pos_x = pos[:, 1:2] / PDE_L
    pos_t = pos[:, 0:1] / PDE_TMAX
    F = jnp.pad(jnp.concatenate([u, pos_x, pos_t], axis=1),
                ((0, NP - N), (0, 0)))
    src2 = jnp.pad(edge_index[0], (0, EP - E),
                   constant_values=N).reshape(NCHUNKS, CHUNK)
    dst2 = jnp.pad(edge_index[1], (0, EP - E),
                   constant_values=N).reshape(NCHUNKS, CHUNK)
    zN = jnp.zeros((NPS, H), jnp.float32)

    WPh = msg1_W[:, 0:H, :]
    WQh = msg1_W[:, H:2 * H, :]
    WPf = msg1_W[:, 2 * H:2 * H + 52, :]
    WQf = jnp.concatenate(
        [-msg1_W[:, 2 * H:2 * H + 51, :],
         jnp.zeros((NLAYERS, 1, H), jnp.float32)], axis=1)
    Uh = upd1_W[:, 0:H, :]
    Ua = upd1_W[:, H:2 * H, :]
    Uv = upd1_W[:, 2 * H:2 * H + 1, :]

    o_, i_, k_, j_ = np.meshgrid(np.arange(8), np.arange(2), np.arange(16),
                                 np.arange(38), indexing="ij")
    K1 = jnp.zeros((2 * H, 320), jnp.float32).at[
        (i_ * H + 3 * j_ + k_).ravel(), (o_ * 38 + j_).ravel()
    ].set(conv1_W[o_.ravel(), i_.ravel(), k_.ravel()])
    b1r = jnp.zeros((320,), jnp.float32).at[
        (o_ * 38 + j_).ravel()].set(conv1_b[o_.ravel()])
    c_, o2_, j2_, jj_ = np.meshgrid(np.arange(2), np.arange(8),
                                    np.arange(14), np.arange(25),
                                    indexing="ij")
    K2 = jnp.zeros((320, 64), jnp.float32).at[
        (o2_ * 38 + jj_ + j2_).ravel(), (c_ * 25 + jj_).ravel()
    ].add(conv2_W[c_.ravel(), o2_.ravel(), j2_.ravel()])
    b2r = jnp.zeros((64,), jnp.float32).at[
        (c_ * 25 + jj_).ravel()].set(conv2_b[c_.ravel()])
    dtv = jnp.zeros((64,), jnp.float32).at[np.arange(50)].set(
        np.tile(PDE_DT * (np.arange(25) + 1.0), 2).astype(np.float32))
    upad = jnp.pad(u, ((0, NP - N), (0, 64 - 2 * TW)))

    deg2 = _sc_deg(dst2)
    h, P, Q, degi = _enc_call(F, deg2, emb_W1, emb_b1, emb_W2, emb_b2,
                              WPh[0], WPf[0], WQh[0], WQf[0], msg1_b[0])
    for l in range(NLAYERS):
        t = _sc_gather(P, Q, dst2, src2)
        m2 = _msg_call(t, msg2_W[l], msg2_b[l])
        agg2 = _sc_scatter(m2, dst2, zN)
        if l < NLAYERS - 1:
            h, P, Q = _upd_call(h, F, agg2, degi,
                                Uh[l], Ua[l], Uv[l], upd1_b[l],
                                upd2_W[l], upd2_b[l],
                                WPh[l + 1], WPf[l + 1], WQh[l + 1],
                                WQf[l + 1], msg1_b[l + 1])
        else:
            outp = _fin_call(h, F, agg2, degi,
                             Uh[l], Ua[l], Uv[l], upd1_b[l],
                             upd2_W[l], upd2_b[l],
                             dbl_W, dbl_b, K1, b1r, K2, b2r, upad, dtv)
    return outp[:N, :2 * TW]


# speedup vs baseline: 4.2392x; 1.2223x over previous
import functools

import jax
import jax.numpy as jnp
import numpy as np
from jax import lax
from jax.experimental import pallas as pl
from jax.experimental.pallas import tpu as pltpu
from jax.experimental.pallas import tpu_sc as plsc

N = 10000
E = 160000
TW = 25
H = 128
NLAYERS = 6
PDE_L = 16.0
PDE_TMAX = 4.0
PDE_DT = 0.01

_INTERPRET = False

NP = 10240
EP = 163840
CHUNK = 128
NCHUNKS = EP // CHUNK
NWORK = 32
CPW = NCHUNKS // NWORK
NPS = NP // 16
NBUF = 2
NGRP = CPW // NBUF
NBUF_S = 2
NGRP_S = CPW // NBUF_S


def _swish(v):
    return v * jax.nn.sigmoid(v)


def _enc_body(F_ref, deg2_ref, eW1_ref, eb1_ref, eW2_ref, eb2_ref,
              WPh_ref, WPf_ref, WQh_ref, WQf_ref, b1_ref,
              h_ref, P_ref, Q_ref, degi_ref):
    F = F_ref[...]
    h = _swish(jnp.dot(F, eW1_ref[...], preferred_element_type=jnp.float32)
               + eb1_ref[...])
    h = _swish(jnp.dot(h, eW2_ref[...], preferred_element_type=jnp.float32)
               + eb2_ref[...])
    h_ref[...] = h
    P_ref[...] = (jnp.dot(h, WPh_ref[...], preferred_element_type=jnp.float32)
                  + jnp.dot(F, WPf_ref[...], preferred_element_type=jnp.float32)
                  + b1_ref[...])
    Q_ref[...] = (jnp.dot(h, WQh_ref[...], preferred_element_type=jnp.float32)
                  + jnp.dot(F, WQf_ref[...], preferred_element_type=jnp.float32))
    deg = deg2_ref[0, :, 0:8] + deg2_ref[1, :, 0:8]
    degi_ref[...] = 1.0 / jnp.maximum(deg, 1.0)


def _enc_call(F, deg2, eW1, eb1, eW2, eb2, WPh, WPf, WQh, WQf, b1):
    return pl.pallas_call(
        _enc_body,
        out_shape=(
            jax.ShapeDtypeStruct((NP, H), jnp.float32),
            jax.ShapeDtypeStruct((NP, H), jnp.float32),
            jax.ShapeDtypeStruct((NP, H), jnp.float32),
            jax.ShapeDtypeStruct((NP, 8), jnp.float32),
        ),
        interpret=_INTERPRET,
    )(F, deg2, eW1, eb1, eW2, eb2, WPh, WPf, WQh, WQf, b1)


def _msg_body(t_ref, W2_ref, b2_ref, m2_ref):
    m = _swish(t_ref[...])
    m2_ref[...] = _swish(
        jnp.dot(m, W2_ref[...], preferred_element_type=jnp.float32)
        + b2_ref[...])


def _msg_call(t, W2, b2):
    blk = 2048
    grid = EP // blk
    return pl.pallas_call(
        _msg_body,
        grid=(grid,),
        in_specs=[
            pl.BlockSpec((blk, H), lambda i: (i, 0)),
            pl.BlockSpec((H, H), lambda i: (0, 0)),
            pl.BlockSpec((H,), lambda i: (0,)),
        ],
        out_specs=pl.BlockSpec((blk, H), lambda i: (i, 0)),
        out_shape=jax.ShapeDtypeStruct((EP, H), jnp.float32),
        interpret=_INTERPRET,
    )(t, W2, b2)


def _upd_common(h_ref, F_ref, agg2_ref, degi_ref,
                Uh_ref, Ua_ref, Uv_ref, ub1_ref, U2_ref, ub2_ref):
    h = h_ref[...]
    agg = (agg2_ref[0] + agg2_ref[1]) * degi_ref[:, 0:1]
    pt = F_ref[:, 51:52]
    u1 = _swish(jnp.dot(h, Uh_ref[...], preferred_element_type=jnp.float32)
                + jnp.dot(agg, Ua_ref[...], preferred_element_type=jnp.float32)
                + pt * Uv_ref[...]
                + ub1_ref[...])
    u2 = _swish(jnp.dot(u1, U2_ref[...], preferred_element_type=jnp.float32)
                + ub2_ref[...])
    hn = h + u2
    mean = jnp.mean(hn[:N], axis=0, keepdims=True)
    hc = hn - mean
    var = jnp.mean(hc[:N] * hc[:N], axis=0, keepdims=True)
    return hc * jax.lax.rsqrt(var + 1e-5)


def _upd_body(h_ref, F_ref, agg2_ref, degi_ref,
              Uh_ref, Ua_ref, Uv_ref, ub1_ref, U2_ref, ub2_ref,
              WPh_ref, WPf_ref, WQh_ref, WQf_ref, b1_ref,
              h_out, P_ref, Q_ref):
    hn = _upd_common(h_ref, F_ref, agg2_ref, degi_ref,
                     Uh_ref, Ua_ref, Uv_ref, ub1_ref, U2_ref, ub2_ref)
    F = F_ref[...]
    h_out[...] = hn
    P_ref[...] = (jnp.dot(hn, WPh_ref[...], preferred_element_type=jnp.float32)
                  + jnp.dot(F, WPf_ref[...], preferred_element_type=jnp.float32)
                  + b1_ref[...])
    Q_ref[...] = (jnp.dot(hn, WQh_ref[...], preferred_element_type=jnp.float32)
                  + jnp.dot(F, WQf_ref[...], preferred_element_type=jnp.float32))


def _upd_call(h, F, agg2, degi, Uh, Ua, Uv, ub1, U2, ub2,
              WPh, WPf, WQh, WQf, b1):
    return pl.pallas_call(
        _upd_body,
        out_shape=(
            jax.ShapeDtypeStruct((NP, H), jnp.float32),
            jax.ShapeDtypeStruct((NP, H), jnp.float32),
            jax.ShapeDtypeStruct((NP, H), jnp.float32),
        ),
        interpret=_INTERPRET,
    )(h, F, agg2, degi, Uh, Ua, Uv, ub1, U2, ub2, WPh, WPf, WQh, WQf, b1)


def _fin_body(h_ref, F_ref, agg2_ref, degi_ref,
              Uh_ref, Ua_ref, Uv_ref, ub1_ref, U2_ref, ub2_ref,
              dblW_ref, dblb_ref, K1_ref, b1r_ref, K2_ref, b2r_ref,
              upad_ref, dtv_ref, out_ref):
    hn = _upd_common(h_ref, F_ref, agg2_ref, degi_ref,
                     Uh_ref, Ua_ref, Uv_ref, ub1_ref, U2_ref, ub2_ref)
    h2 = _swish(jnp.dot(hn, dblW_ref[...], preferred_element_type=jnp.float32)
                + dblb_ref[...])
    c = _swish(jnp.dot(h2, K1_ref[...], preferred_element_type=jnp.float32)
               + b1r_ref[...])
    diff = (jnp.dot(c, K2_ref[...], preferred_element_type=jnp.float32)
            + b2r_ref[...])
    out_ref[...] = upad_ref[...] + dtv_ref[...] * diff


def _fin_call(h, F, agg2, degi, Uh, Ua, Uv, ub1, U2, ub2,
              dblW, dblb, K1, b1r, K2, b2r, upad, dtv):
    return pl.pallas_call(
        _fin_body,
        out_shape=jax.ShapeDtypeStruct((NP, 64), jnp.float32),
        interpret=_INTERPRET,
    )(h, F, agg2, degi, Uh, Ua, Uv, ub1, U2, ub2,
      dblW, dblb, K1, b1r, K2, b2r, upad, dtv)


_MESH = plsc.VectorSubcoreMesh(core_axis_name="c", subcore_axis_name="s")


def _wid():
    return lax.axis_index("s") * 2 + lax.axis_index("c")


@functools.partial(
    pl.kernel,
    out_type=jax.ShapeDtypeStruct((EP, H), jnp.float32),
    mesh=_MESH,
    scratch_types=[
        pltpu.VMEM((CPW, CHUNK), jnp.int32),
        pltpu.VMEM((CPW, CHUNK), jnp.int32),
        pltpu.VMEM((NBUF, CHUNK, H), jnp.float32),
        pltpu.VMEM_SHARED((NP, H), jnp.float32),
        pltpu.SemaphoreType.DMA,
        pltpu.SemaphoreType.DMA,
    ],
)
def _sc_gather_kernel(P_hbm, Q_hbm, dst2_hbm, src2_hbm, t_hbm,
                      idxd_v, idxs_v, rows_v, P_sh, s0, s1):
    w = _wid()
    s = lax.axis_index("s")
    sems = (s0, s1)
    pltpu.sync_copy(P_hbm.at[pl.ds(s * NPS, NPS)],
                    P_sh.at[pl.ds(s * NPS, NPS)])
    pltpu.sync_copy(dst2_hbm.at[pl.ds(w * CPW, CPW)], idxd_v)
    pltpu.sync_copy(src2_hbm.at[pl.ds(w * CPW, CPW)], idxs_v)
    plsc.subcore_barrier()

    def group(g, _):
        cps = [pltpu.async_copy(P_sh.at[idxd_v.at[g * NBUF + b]],
                                rows_v.at[b], sems[b])
               for b in range(NBUF)]
        cqs = []
        for b in range(NBUF):
            cps[b].wait()
            cqs.append(pltpu.async_copy(Q_hbm.at[idxs_v.at[g * NBUF + b]],
                                        rows_v.at[b], sems[b], add=True))
        cws = []
        for b in range(NBUF):
            cqs[b].wait()
            row = w * CPW + g * NBUF + b
            cws.append(pltpu.async_copy(rows_v.at[b],
                                        t_hbm.at[pl.ds(row * CHUNK, CHUNK)],
                                        sems[b]))
        for b in range(NBUF):
            cws[b].wait()
        return 0

    lax.fori_loop(0, NGRP, group, 0)


def _sc_gather(P, Q, dst2, src2):
    return _sc_gather_kernel(P, Q, dst2, src2)


@functools.partial(
    pl.kernel,
    out_type=jax.ShapeDtypeStruct((2, NP, H), jnp.float32),
    mesh=_MESH,
    scratch_types=[
        pltpu.VMEM((CPW, CHUNK), jnp.int32),
        pltpu.VMEM((NBUF_S, CHUNK, H), jnp.float32),
        pltpu.VMEM_SHARED((NP, H), jnp.float32),
        pltpu.SemaphoreType.DMA,
        pltpu.SemaphoreType.DMA,
    ],
)
def _sc_scatter_kernel(m2_hbm, dst2_hbm, z_hbm, agg2_hbm,
                       idx_v, buf_v, agg_sh, s0, s1):
    c = lax.axis_index("c")
    s = lax.axis_index("s")
    sems = (s0, s1)
    pltpu.sync_copy(z_hbm, agg_sh.at[pl.ds(s * NPS, NPS)])
    plsc.subcore_barrier()
    w = _wid()
    pltpu.sync_copy(dst2_hbm.at[pl.ds(w * CPW, CPW)], idx_v)

    def group(g, _):
        crs = [pltpu.async_copy(
                   m2_hbm.at[pl.ds((w * CPW + g * NBUF_S + b) * CHUNK, CHUNK)],
                   buf_v.at[b], sems[b])
               for b in range(NBUF_S)]
        css = []
        for b in range(NBUF_S):
            crs[b].wait()
            css.append(pltpu.async_copy(buf_v.at[b],
                                        agg_sh.at[idx_v.at[g * NBUF_S + b]],
                                        sems[b], add=True))
        for b in range(NBUF_S):
            css[b].wait()
        return 0

    lax.fori_loop(0, NGRP_S, group, 0)
    plsc.subcore_barrier()
    pltpu.sync_copy(agg_sh.at[pl.ds(s * NPS, NPS)],
                    agg2_hbm.at[c, pl.ds(s * NPS, NPS)])


def _sc_scatter(m2, dst2, z):
    return _sc_scatter_kernel(m2, dst2, z)


@functools.partial(
    pl.kernel,
    out_type=jax.ShapeDtypeStruct((2, NP, H), jnp.float32),
    mesh=_MESH,
    scratch_types=[
        pltpu.VMEM((CPW, CHUNK), jnp.int32),
        pltpu.VMEM((CHUNK, H), jnp.float32),
        pltpu.VMEM_SHARED((NP, H), jnp.float32),
        pltpu.SemaphoreType.DMA,
        pltpu.SemaphoreType.DMA,
        pltpu.SemaphoreType.DMA,
        pltpu.SemaphoreType.DMA,
    ],
)
def _sc_deg_kernel(dst2_hbm, ones_hbm, z16_hbm, deg2_hbm,
                   idx_v, ones_v, deg_sh, s0, s1, s2, s3):
    c = lax.axis_index("c")
    s = lax.axis_index("s")
    sems = (s0, s1, s2, s3)
    pltpu.sync_copy(z16_hbm, deg_sh.at[pl.ds(s * NPS, NPS)])
    pltpu.sync_copy(ones_hbm, ones_v)
    plsc.subcore_barrier()
    w = _wid()
    pltpu.sync_copy(dst2_hbm.at[pl.ds(w * CPW, CPW)], idx_v)

    def group(g, _):
        cs = [pltpu.async_copy(ones_v, deg_sh.at[idx_v.at[g * 4 + b]],
                               sems[b], add=True)
              for b in range(4)]
        for b in range(4):
            cs[b].wait()
        return 0

    lax.fori_loop(0, CPW // 4, group, 0)
    plsc.subcore_barrier()
    pltpu.sync_copy(deg_sh.at[pl.ds(s * NPS, NPS)],
                    deg2_hbm.at[c, pl.ds(s * NPS, NPS)])


def _sc_deg(dst2):
    ones = jnp.ones((CHUNK, H), jnp.float32)
    z16 = jnp.zeros((NPS, H), jnp.float32)
    return _sc_deg_kernel(dst2, ones, z16)


def kernel(x, pos, edge_index, batch, emb_W1, emb_b1, emb_W2, emb_b2,
           msg1_W, msg1_b, msg2_W, msg2_b, upd1_W, upd1_b, upd2_W, upd2_b,
           dbl_W, dbl_b, conv1_W, conv1_b, conv2_W, conv2_b):
    u = x
    pos_x = pos[:, 1:2] / PDE_L
    pos_t = pos[:, 0:1] / PDE_TMAX
    F = jnp.pad(jnp.concatenate([u, pos_x, pos_t], axis=1),
                ((0, NP - N), (0, 0)))
    src2 = jnp.pad(edge_index[0], (0, EP - E),
                   constant_values=N).reshape(NCHUNKS, CHUNK)
    dst2 = jnp.pad(edge_index[1], (0, EP - E),
                   constant_values=N).reshape(NCHUNKS, CHUNK)
    zN = jnp.zeros((NPS, H), jnp.float32)

    WPh = msg1_W[:, 0:H, :]
    WQh = msg1_W[:, H:2 * H, :]
    WPf = msg1_W[:, 2 * H:2 * H + 52, :]
    WQf = jnp.concatenate(
        [-msg1_W[:, 2 * H:2 * H + 51, :],
         jnp.zeros((NLAYERS, 1, H), jnp.float32)], axis=1)
    Uh = upd1_W[:, 0:H, :]
    Ua = upd1_W[:, H:2 * H, :]
    Uv = upd1_W[:, 2 * H:2 * H + 1, :]

    o_, i_, k_, j_ = np.meshgrid(np.arange(8), np.arange(2), np.arange(16),
                                 np.arange(38), indexing="ij")
    K1 = jnp.zeros((2 * H, 320), jnp.float32).at[
        (i_ * H + 3 * j_ + k_).ravel(), (o_ * 38 + j_).ravel()
    ].set(conv1_W[o_.ravel(), i_.ravel(), k_.ravel()])
    b1r = jnp.zeros((320,), jnp.float32).at[
        (o_ * 38 + j_).ravel()].set(conv1_b[o_.ravel()])
    c_, o2_, j2_, jj_ = np.meshgrid(np.arange(2), np.arange(8),
                                    np.arange(14), np.arange(25),
                                    indexing="ij")
    K2 = jnp.zeros((320, 64), jnp.float32).at[
        (o2_ * 38 + jj_ + j2_).ravel(), (c_ * 25 + jj_).ravel()
    ].add(conv2_W[c_.ravel(), o2_.ravel(), j2_.ravel()])
    b2r = jnp.zeros((64,), jnp.float32).at[
        (c_ * 25 + jj_).ravel()].set(conv2_b[c_.ravel()])
    dtv = jnp.zeros((64,), jnp.float32).at[np.arange(50)].set(
        np.tile(PDE_DT * (np.arange(25) + 1.0), 2).astype(np.float32))
    upad = jnp.pad(u, ((0, NP - N), (0, 64 - 2 * TW)))

    deg2 = _sc_deg(dst2)
    h, P, Q, degi = _enc_call(F, deg2, emb_W1, emb_b1, emb_W2, emb_b2,
                              WPh[0], WPf[0], WQh[0], WQf[0], msg1_b[0])
    for l in range(NLAYERS):
        t = _sc_gather(P, Q, dst2, src2)
        m2 = _msg_call(t, msg2_W[l], msg2_b[l])
        agg2 = _sc_scatter(m2, dst2, zN)
        if l < NLAYERS - 1:
            h, P, Q = _upd_call(h, F, agg2, degi,
                                Uh[l], Ua[l], Uv[l], upd1_b[l],
                                upd2_W[l], upd2_b[l],
                                WPh[l + 1], WPf[l + 1], WQh[l + 1],
                                WQf[l + 1], msg1_b[l + 1])
        else:
            outp = _fin_call(h, F, agg2, degi,
                             Uh[l], Ua[l], Uv[l], upd1_b[l],
                             upd2_W[l], upd2_b[l],
                             dbl_W, dbl_b, K1, b1r, K2, b2r, upad, dtv)
    return outp[:N, :2 * TW]


# speedup vs baseline: 4.2420x; 1.0007x over previous
import functools

import jax
import jax.numpy as jnp
import numpy as np
from jax import lax
from jax.experimental import pallas as pl
from jax.experimental.pallas import tpu as pltpu
from jax.experimental.pallas import tpu_sc as plsc

N = 10000
E = 160000
TW = 25
H = 128
NLAYERS = 6
PDE_L = 16.0
PDE_TMAX = 4.0
PDE_DT = 0.01

_INTERPRET = False

NP = 10240
EP = 163840
CHUNK = 128
NCHUNKS = EP // CHUNK
NWORK = 32
CPW = NCHUNKS // NWORK
NPS = NP // 16
NBUF = 2
NGRP = CPW // NBUF
NBUF_S = 2
NGRP_S = CPW // NBUF_S


def _swish(v):
    return v * jax.nn.sigmoid(v)


def _enc_body(F_ref, deg2_ref, eW1_ref, eb1_ref, eW2_ref, eb2_ref,
              WPh_ref, WPf_ref, WQh_ref, WQf_ref, b1_ref,
              h_ref, P_ref, Q_ref, degi_ref):
    F = F_ref[...]
    h = _swish(jnp.dot(F, eW1_ref[...], preferred_element_type=jnp.float32)
               + eb1_ref[...])
    h = _swish(jnp.dot(h, eW2_ref[...], preferred_element_type=jnp.float32)
               + eb2_ref[...])
    h_ref[...] = h
    P_ref[...] = (jnp.dot(h, WPh_ref[...], preferred_element_type=jnp.float32)
                  + jnp.dot(F, WPf_ref[...], preferred_element_type=jnp.float32)
                  + b1_ref[...])
    Q_ref[...] = (jnp.dot(h, WQh_ref[...], preferred_element_type=jnp.float32)
                  + jnp.dot(F, WQf_ref[...], preferred_element_type=jnp.float32))
    deg = deg2_ref[0, :, 0:8] + deg2_ref[1, :, 0:8]
    degi_ref[...] = 1.0 / jnp.maximum(deg, 1.0)


def _enc_call(F, deg2, eW1, eb1, eW2, eb2, WPh, WPf, WQh, WQf, b1):
    return pl.pallas_call(
        _enc_body,
        out_shape=(
            jax.ShapeDtypeStruct((NP, H), jnp.float32),
            jax.ShapeDtypeStruct((NP, H), jnp.float32),
            jax.ShapeDtypeStruct((NP, H), jnp.float32),
            jax.ShapeDtypeStruct((NP, 8), jnp.float32),
        ),
        interpret=_INTERPRET,
    )(F, deg2, eW1, eb1, eW2, eb2, WPh, WPf, WQh, WQf, b1)


def _msg_body(t_ref, W2_ref, b2_ref, m2_ref):
    m = _swish(t_ref[...]).astype(jnp.bfloat16)
    m2_ref[...] = _swish(
        jnp.dot(m, W2_ref[...].astype(jnp.bfloat16),
                preferred_element_type=jnp.float32)
        + b2_ref[...])


def _msg_call(t, W2, b2):
    blk = 2048
    grid = EP // blk
    return pl.pallas_call(
        _msg_body,
        grid=(grid,),
        in_specs=[
            pl.BlockSpec((blk, H), lambda i: (i, 0)),
            pl.BlockSpec((H, H), lambda i: (0, 0)),
            pl.BlockSpec((H,), lambda i: (0,)),
        ],
        out_specs=pl.BlockSpec((blk, H), lambda i: (i, 0)),
        out_shape=jax.ShapeDtypeStruct((EP, H), jnp.float32),
        interpret=_INTERPRET,
    )(t, W2, b2)


def _upd_common(h_ref, F_ref, agg2_ref, degi_ref,
                Uh_ref, Ua_ref, Uv_ref, ub1_ref, U2_ref, ub2_ref):
    h = h_ref[...]
    agg = (agg2_ref[0] + agg2_ref[1]) * degi_ref[:, 0:1]
    pt = F_ref[:, 51:52]
    u1 = _swish(jnp.dot(h, Uh_ref[...], preferred_element_type=jnp.float32)
                + jnp.dot(agg, Ua_ref[...], preferred_element_type=jnp.float32)
                + pt * Uv_ref[...]
                + ub1_ref[...])
    u2 = _swish(jnp.dot(u1, U2_ref[...], preferred_element_type=jnp.float32)
                + ub2_ref[...])
    hn = h + u2
    mean = jnp.mean(hn[:N], axis=0, keepdims=True)
    hc = hn - mean
    var = jnp.mean(hc[:N] * hc[:N], axis=0, keepdims=True)
    return hc * jax.lax.rsqrt(var + 1e-5)


def _upd_body(h_ref, F_ref, agg2_ref, degi_ref,
              Uh_ref, Ua_ref, Uv_ref, ub1_ref, U2_ref, ub2_ref,
              WPh_ref, WPf_ref, WQh_ref, WQf_ref, b1_ref,
              h_out, P_ref, Q_ref):
    hn = _upd_common(h_ref, F_ref, agg2_ref, degi_ref,
                     Uh_ref, Ua_ref, Uv_ref, ub1_ref, U2_ref, ub2_ref)
    F = F_ref[...]
    h_out[...] = hn
    P_ref[...] = (jnp.dot(hn, WPh_ref[...], preferred_element_type=jnp.float32)
                  + jnp.dot(F, WPf_ref[...], preferred_element_type=jnp.float32)
                  + b1_ref[...])
    Q_ref[...] = (jnp.dot(hn, WQh_ref[...], preferred_element_type=jnp.float32)
                  + jnp.dot(F, WQf_ref[...], preferred_element_type=jnp.float32))


def _upd_call(h, F, agg2, degi, Uh, Ua, Uv, ub1, U2, ub2,
              WPh, WPf, WQh, WQf, b1):
    return pl.pallas_call(
        _upd_body,
        out_shape=(
            jax.ShapeDtypeStruct((NP, H), jnp.float32),
            jax.ShapeDtypeStruct((NP, H), jnp.float32),
            jax.ShapeDtypeStruct((NP, H), jnp.float32),
        ),
        interpret=_INTERPRET,
    )(h, F, agg2, degi, Uh, Ua, Uv, ub1, U2, ub2, WPh, WPf, WQh, WQf, b1)


def _fin_body(h_ref, F_ref, agg2_ref, degi_ref,
              Uh_ref, Ua_ref, Uv_ref, ub1_ref, U2_ref, ub2_ref,
              dblW_ref, dblb_ref, K1_ref, b1r_ref, K2_ref, b2r_ref,
              upad_ref, dtv_ref, out_ref):
    hn = _upd_common(h_ref, F_ref, agg2_ref, degi_ref,
                     Uh_ref, Ua_ref, Uv_ref, ub1_ref, U2_ref, ub2_ref)
    h2 = _swish(jnp.dot(hn, dblW_ref[...], preferred_element_type=jnp.float32)
                + dblb_ref[...])
    c = _swish(jnp.dot(h2, K1_ref[...], preferred_element_type=jnp.float32)
               + b1r_ref[...])
    diff = (jnp.dot(c, K2_ref[...], preferred_element_type=jnp.float32)
            + b2r_ref[...])
    out_ref[...] = upad_ref[...] + dtv_ref[...] * diff


def _fin_call(h, F, agg2, degi, Uh, Ua, Uv, ub1, U2, ub2,
              dblW, dblb, K1, b1r, K2, b2r, upad, dtv):
    return pl.pallas_call(
        _fin_body,
        out_shape=jax.ShapeDtypeStruct((NP, 64), jnp.float32),
        interpret=_INTERPRET,
    )(h, F, agg2, degi, Uh, Ua, Uv, ub1, U2, ub2,
      dblW, dblb, K1, b1r, K2, b2r, upad, dtv)


_MESH = plsc.VectorSubcoreMesh(core_axis_name="c", subcore_axis_name="s")


def _wid():
    return lax.axis_index("s") * 2 + lax.axis_index("c")


@functools.partial(
    pl.kernel,
    out_type=jax.ShapeDtypeStruct((EP, H), jnp.float32),
    mesh=_MESH,
    scratch_types=[
        pltpu.VMEM((CPW, CHUNK), jnp.int32),
        pltpu.VMEM((CPW, CHUNK), jnp.int32),
        pltpu.VMEM((NBUF, CHUNK, H), jnp.float32),
        pltpu.VMEM_SHARED((NP, H), jnp.float32),
        pltpu.SemaphoreType.DMA,
        pltpu.SemaphoreType.DMA,
    ],
)
def _sc_gather_kernel(P_hbm, Q_hbm, dst2_hbm, src2_hbm, t_hbm,
                      idxd_v, idxs_v, rows_v, P_sh, s0, s1):
    w = _wid()
    s = lax.axis_index("s")
    sems = (s0, s1)
    pltpu.sync_copy(P_hbm.at[pl.ds(s * NPS, NPS)],
                    P_sh.at[pl.ds(s * NPS, NPS)])
    pltpu.sync_copy(dst2_hbm.at[pl.ds(w * CPW, CPW)], idxd_v)
    pltpu.sync_copy(src2_hbm.at[pl.ds(w * CPW, CPW)], idxs_v)
    plsc.subcore_barrier()

    def group(g, _):
        cps = [pltpu.async_copy(P_sh.at[idxd_v.at[g * NBUF + b]],
                                rows_v.at[b], sems[b])
               for b in range(NBUF)]
        cqs = []
        for b in range(NBUF):
            cps[b].wait()
            cqs.append(pltpu.async_copy(Q_hbm.at[idxs_v.at[g * NBUF + b]],
                                        rows_v.at[b], sems[b], add=True))
        cws = []
        for b in range(NBUF):
            cqs[b].wait()
            row = w * CPW + g * NBUF + b
            cws.append(pltpu.async_copy(rows_v.at[b],
                                        t_hbm.at[pl.ds(row * CHUNK, CHUNK)],
                                        sems[b]))
        for b in range(NBUF):
            cws[b].wait()
        return 0

    lax.fori_loop(0, NGRP, group, 0)


def _sc_gather(P, Q, dst2, src2):
    return _sc_gather_kernel(P, Q, dst2, src2)


@functools.partial(
    pl.kernel,
    out_type=jax.ShapeDtypeStruct((2, NP, H), jnp.float32),
    mesh=_MESH,
    scratch_types=[
        pltpu.VMEM((CPW, CHUNK), jnp.int32),
        pltpu.VMEM((NBUF_S, CHUNK, H), jnp.float32),
        pltpu.VMEM_SHARED((NP, H), jnp.float32),
        pltpu.SemaphoreType.DMA,
        pltpu.SemaphoreType.DMA,
    ],
)
def _sc_scatter_kernel(m2_hbm, dst2_hbm, z_hbm, agg2_hbm,
                       idx_v, buf_v, agg_sh, s0, s1):
    c = lax.axis_index("c")
    s = lax.axis_index("s")
    sems = (s0, s1)
    pltpu.sync_copy(z_hbm, agg_sh.at[pl.ds(s * NPS, NPS)])
    plsc.subcore_barrier()
    w = _wid()
    pltpu.sync_copy(dst2_hbm.at[pl.ds(w * CPW, CPW)], idx_v)

    def group(g, _):
        crs = [pltpu.async_copy(
                   m2_hbm.at[pl.ds((w * CPW + g * NBUF_S + b) * CHUNK, CHUNK)],
                   buf_v.at[b], sems[b])
               for b in range(NBUF_S)]
        css = []
        for b in range(NBUF_S):
            crs[b].wait()
            css.append(pltpu.async_copy(buf_v.at[b],
                                        agg_sh.at[idx_v.at[g * NBUF_S + b]],
                                        sems[b], add=True))
        for b in range(NBUF_S):
            css[b].wait()
        return 0

    lax.fori_loop(0, NGRP_S, group, 0)
    plsc.subcore_barrier()
    pltpu.sync_copy(agg_sh.at[pl.ds(s * NPS, NPS)],
                    agg2_hbm.at[c, pl.ds(s * NPS, NPS)])


def _sc_scatter(m2, dst2, z):
    return _sc_scatter_kernel(m2, dst2, z)


@functools.partial(
    pl.kernel,
    out_type=jax.ShapeDtypeStruct((2, NP, H), jnp.float32),
    mesh=_MESH,
    scratch_types=[
        pltpu.VMEM((CPW, CHUNK), jnp.int32),
        pltpu.VMEM((CHUNK, H), jnp.float32),
        pltpu.VMEM_SHARED((NP, H), jnp.float32),
        pltpu.SemaphoreType.DMA,
        pltpu.SemaphoreType.DMA,
        pltpu.SemaphoreType.DMA,
        pltpu.SemaphoreType.DMA,
    ],
)
def _sc_deg_kernel(dst2_hbm, ones_hbm, z16_hbm, deg2_hbm,
                   idx_v, ones_v, deg_sh, s0, s1, s2, s3):
    c = lax.axis_index("c")
    s = lax.axis_index("s")
    sems = (s0, s1, s2, s3)
    pltpu.sync_copy(z16_hbm, deg_sh.at[pl.ds(s * NPS, NPS)])
    pltpu.sync_copy(ones_hbm, ones_v)
    plsc.subcore_barrier()
    w = _wid()
    pltpu.sync_copy(dst2_hbm.at[pl.ds(w * CPW, CPW)], idx_v)

    def group(g, _):
        cs = [pltpu.async_copy(ones_v, deg_sh.at[idx_v.at[g * 4 + b]],
                               sems[b], add=True)
              for b in range(4)]
        for b in range(4):
            cs[b].wait()
        return 0

    lax.fori_loop(0, CPW // 4, group, 0)
    plsc.subcore_barrier()
    pltpu.sync_copy(deg_sh.at[pl.ds(s * NPS, NPS)],
                    deg2_hbm.at[c, pl.ds(s * NPS, NPS)])


def _sc_deg(dst2):
    ones = jnp.ones((CHUNK, H), jnp.float32)
    z16 = jnp.zeros((NPS, H), jnp.float32)
    return _sc_deg_kernel(dst2, ones, z16)


def kernel(x, pos, edge_index, batch, emb_W1, emb_b1, emb_W2, emb_b2,
           msg1_W, msg1_b, msg2_W, msg2_b, upd1_W, upd1_b, upd2_W, upd2_b,
           dbl_W, dbl_b, conv1_W, conv1_b, conv2_W, conv2_b):
    u = x
    pos_x = pos[:, 1:2] / PDE_L
    pos_t = pos[:, 0:1] / PDE_TMAX
    F = jnp.pad(jnp.concatenate([u, pos_x, pos_t], axis=1),
                ((0, NP - N), (0, 0)))
    src2 = jnp.pad(edge_index[0], (0, EP - E),
                   constant_values=N).reshape(NCHUNKS, CHUNK)
    dst2 = jnp.pad(edge_index[1], (0, EP - E),
                   constant_values=N).reshape(NCHUNKS, CHUNK)
    zN = jnp.zeros((NPS, H), jnp.float32)

    WPh = msg1_W[:, 0:H, :]
    WQh = msg1_W[:, H:2 * H, :]
    WPf = msg1_W[:, 2 * H:2 * H + 52, :]
    WQf = jnp.concatenate(
        [-msg1_W[:, 2 * H:2 * H + 51, :],
         jnp.zeros((NLAYERS, 1, H), jnp.float32)], axis=1)
    Uh = upd1_W[:, 0:H, :]
    Ua = upd1_W[:, H:2 * H, :]
    Uv = upd1_W[:, 2 * H:2 * H + 1, :]

    o_, i_, k_, j_ = np.meshgrid(np.arange(8), np.arange(2), np.arange(16),
                                 np.arange(38), indexing="ij")
    K1 = jnp.zeros((2 * H, 320), jnp.float32).at[
        (i_ * H + 3 * j_ + k_).ravel(), (o_ * 38 + j_).ravel()
    ].set(conv1_W[o_.ravel(), i_.ravel(), k_.ravel()])
    b1r = jnp.zeros((320,), jnp.float32).at[
        (o_ * 38 + j_).ravel()].set(conv1_b[o_.ravel()])
    c_, o2_, j2_, jj_ = np.meshgrid(np.arange(2), np.arange(8),
                                    np.arange(14), np.arange(25),
                                    indexing="ij")
    K2 = jnp.zeros((320, 64), jnp.float32).at[
        (o2_ * 38 + jj_ + j2_).ravel(), (c_ * 25 + jj_).ravel()
    ].add(conv2_W[c_.ravel(), o2_.ravel(), j2_.ravel()])
    b2r = jnp.zeros((64,), jnp.float32).at[
        (c_ * 25 + jj_).ravel()].set(conv2_b[c_.ravel()])
    dtv = jnp.zeros((64,), jnp.float32).at[np.arange(50)].set(
        np.tile(PDE_DT * (np.arange(25) + 1.0), 2).astype(np.float32))
    upad = jnp.pad(u, ((0, NP - N), (0, 64 - 2 * TW)))

    deg2 = _sc_deg(dst2)
    h, P, Q, degi = _enc_call(F, deg2, emb_W1, emb_b1, emb_W2, emb_b2,
                              WPh[0], WPf[0], WQh[0], WQf[0], msg1_b[0])
    for l in range(NLAYERS):
        t = _sc_gather(P, Q, dst2, src2)
        m2 = _msg_call(t, msg2_W[l], msg2_b[l])
        agg2 = _sc_scatter(m2, dst2, zN)
        if l < NLAYERS - 1:
            h, P, Q = _upd_call(h, F, agg2, degi,
                                Uh[l], Ua[l], Uv[l], upd1_b[l],
                                upd2_W[l], upd2_b[l],
                                WPh[l + 1], WPf[l + 1], WQh[l + 1],
                                WQf[l + 1], msg1_b[l + 1])
        else:
            outp = _fin_call(h, F, agg2, degi,
                             Uh[l], Ua[l], Uv[l], upd1_b[l],
                             upd2_W[l], upd2_b[l],
                             dbl_W, dbl_b, K1, b1r, K2, b2r, upad, dtv)
    return outp[:N, :2 * TW]


# speedup vs baseline: 4.2999x; 1.0136x over previous
import functools

import jax
import jax.numpy as jnp
import numpy as np
from jax import lax
from jax.experimental import pallas as pl
from jax.experimental.pallas import tpu as pltpu
from jax.experimental.pallas import tpu_sc as plsc

N = 10000
E = 160000
TW = 25
H = 128
NLAYERS = 6
PDE_L = 16.0
PDE_TMAX = 4.0
PDE_DT = 0.01

_INTERPRET = False

NP = 10240
EP = 163840
CHUNK = 128
NCHUNKS = EP // CHUNK
NWORK = 32
CPW = NCHUNKS // NWORK
NPS = NP // 16
NBUF = 3
NGRP = CPW // NBUF
NPG = 10112
NBUF_S = 2
NGRP_S = CPW // NBUF_S


def _swish(v):
    return v * jax.nn.sigmoid(v)


def _enc_body(F_ref, deg2_ref, eW1_ref, eb1_ref, eW2_ref, eb2_ref,
              WPh_ref, WPf_ref, WQh_ref, WQf_ref, b1_ref,
              h_ref, P_ref, Q_ref, degi_ref):
    F = F_ref[...]
    h = _swish(jnp.dot(F, eW1_ref[...], preferred_element_type=jnp.float32)
               + eb1_ref[...])
    h = _swish(jnp.dot(h, eW2_ref[...], preferred_element_type=jnp.float32)
               + eb2_ref[...])
    h_ref[...] = h
    P_ref[...] = (jnp.dot(h, WPh_ref[...], preferred_element_type=jnp.float32)
                  + jnp.dot(F, WPf_ref[...], preferred_element_type=jnp.float32)
                  + b1_ref[...])
    Q_ref[...] = (jnp.dot(h, WQh_ref[...], preferred_element_type=jnp.float32)
                  + jnp.dot(F, WQf_ref[...], preferred_element_type=jnp.float32))
    deg = deg2_ref[0, :, 0:8] + deg2_ref[1, :, 0:8]
    degi_ref[...] = 1.0 / jnp.maximum(deg, 1.0)


def _enc_call(F, deg2, eW1, eb1, eW2, eb2, WPh, WPf, WQh, WQf, b1):
    return pl.pallas_call(
        _enc_body,
        out_shape=(
            jax.ShapeDtypeStruct((NP, H), jnp.float32),
            jax.ShapeDtypeStruct((NP, H), jnp.float32),
            jax.ShapeDtypeStruct((NP, H), jnp.float32),
            jax.ShapeDtypeStruct((NP, 8), jnp.float32),
        ),
        interpret=_INTERPRET,
    )(F, deg2, eW1, eb1, eW2, eb2, WPh, WPf, WQh, WQf, b1)


def _msg_body(t_ref, W2_ref, b2_ref, m2_ref):
    m = _swish(t_ref[...])
    m2_ref[...] = _swish(
        jnp.dot(m, W2_ref[...], preferred_element_type=jnp.float32)
        + b2_ref[...])


def _msg_call(t, W2, b2):
    blk = 2048
    grid = EP // blk
    return pl.pallas_call(
        _msg_body,
        grid=(grid,),
        in_specs=[
            pl.BlockSpec((blk, H), lambda i: (i, 0)),
            pl.BlockSpec((H, H), lambda i: (0, 0)),
            pl.BlockSpec((H,), lambda i: (0,)),
        ],
        out_specs=pl.BlockSpec((blk, H), lambda i: (i, 0)),
        out_shape=jax.ShapeDtypeStruct((EP, H), jnp.float32),
        interpret=_INTERPRET,
    )(t, W2, b2)


def _upd_common(h_ref, F_ref, agg2_ref, degi_ref,
                Uh_ref, Ua_ref, Uv_ref, ub1_ref, U2_ref, ub2_ref):
    h = h_ref[...]
    agg = (agg2_ref[0] + agg2_ref[1]) * degi_ref[:, 0:1]
    pt = F_ref[:, 51:52]
    u1 = _swish(jnp.dot(h, Uh_ref[...], preferred_element_type=jnp.float32)
                + jnp.dot(agg, Ua_ref[...], preferred_element_type=jnp.float32)
                + pt * Uv_ref[...]
                + ub1_ref[...])
    u2 = _swish(jnp.dot(u1, U2_ref[...], preferred_element_type=jnp.float32)
                + ub2_ref[...])
    hn = h + u2
    mean = jnp.mean(hn[:N], axis=0, keepdims=True)
    hc = hn - mean
    var = jnp.mean(hc[:N] * hc[:N], axis=0, keepdims=True)
    return hc * jax.lax.rsqrt(var + 1e-5)


def _upd_body(h_ref, F_ref, agg2_ref, degi_ref,
              Uh_ref, Ua_ref, Uv_ref, ub1_ref, U2_ref, ub2_ref,
              WPh_ref, WPf_ref, WQh_ref, WQf_ref, b1_ref,
              h_out, P_ref, Q_ref):
    hn = _upd_common(h_ref, F_ref, agg2_ref, degi_ref,
                     Uh_ref, Ua_ref, Uv_ref, ub1_ref, U2_ref, ub2_ref)
    F = F_ref[...]
    h_out[...] = hn
    P_ref[...] = (jnp.dot(hn, WPh_ref[...], preferred_element_type=jnp.float32)
                  + jnp.dot(F, WPf_ref[...], preferred_element_type=jnp.float32)
                  + b1_ref[...])
    Q_ref[...] = (jnp.dot(hn, WQh_ref[...], preferred_element_type=jnp.float32)
                  + jnp.dot(F, WQf_ref[...], preferred_element_type=jnp.float32))


def _upd_call(h, F, agg2, degi, Uh, Ua, Uv, ub1, U2, ub2,
              WPh, WPf, WQh, WQf, b1):
    return pl.pallas_call(
        _upd_body,
        out_shape=(
            jax.ShapeDtypeStruct((NP, H), jnp.float32),
            jax.ShapeDtypeStruct((NP, H), jnp.float32),
            jax.ShapeDtypeStruct((NP, H), jnp.float32),
        ),
        interpret=_INTERPRET,
    )(h, F, agg2, degi, Uh, Ua, Uv, ub1, U2, ub2, WPh, WPf, WQh, WQf, b1)


def _fin_body(h_ref, F_ref, agg2_ref, degi_ref,
              Uh_ref, Ua_ref, Uv_ref, ub1_ref, U2_ref, ub2_ref,
              dblW_ref, dblb_ref, K1_ref, b1r_ref, K2_ref, b2r_ref,
              upad_ref, dtv_ref, out_ref):
    hn = _upd_common(h_ref, F_ref, agg2_ref, degi_ref,
                     Uh_ref, Ua_ref, Uv_ref, ub1_ref, U2_ref, ub2_ref)
    h2 = _swish(jnp.dot(hn, dblW_ref[...], preferred_element_type=jnp.float32)
                + dblb_ref[...])
    c = _swish(jnp.dot(h2, K1_ref[...], preferred_element_type=jnp.float32)
               + b1r_ref[...])
    diff = (jnp.dot(c, K2_ref[...], preferred_element_type=jnp.float32)
            + b2r_ref[...])
    out_ref[...] = upad_ref[...] + dtv_ref[...] * diff


def _fin_call(h, F, agg2, degi, Uh, Ua, Uv, ub1, U2, ub2,
              dblW, dblb, K1, b1r, K2, b2r, upad, dtv):
    return pl.pallas_call(
        _fin_body,
        out_shape=jax.ShapeDtypeStruct((NP, 64), jnp.float32),
        interpret=_INTERPRET,
    )(h, F, agg2, degi, Uh, Ua, Uv, ub1, U2, ub2,
      dblW, dblb, K1, b1r, K2, b2r, upad, dtv)


_MESH = plsc.VectorSubcoreMesh(core_axis_name="c", subcore_axis_name="s")


def _wid():
    return lax.axis_index("s") * 2 + lax.axis_index("c")


@functools.partial(
    pl.kernel,
    out_type=jax.ShapeDtypeStruct((EP, H), jnp.float32),
    mesh=_MESH,
    scratch_types=[
        pltpu.VMEM((NBUF, CHUNK), jnp.int32),
        pltpu.VMEM((NBUF, CHUNK), jnp.int32),
        pltpu.VMEM((NBUF, CHUNK, H), jnp.float32),
        pltpu.VMEM_SHARED((NPG, H), jnp.float32),
        pltpu.SemaphoreType.DMA,
        pltpu.SemaphoreType.DMA,
        pltpu.SemaphoreType.DMA,
        pltpu.SemaphoreType.DMA,
        pltpu.SemaphoreType.DMA,
        pltpu.SemaphoreType.DMA,
    ],
)
def _sc_gather_kernel(P_hbm, Q_hbm, dst1_hbm, src1_hbm, t_hbm,
                      idxd_v, idxs_v, rows_v, P_sh, a0, a1, a2, b0, b1, b2):
    w = _wid()
    s = lax.axis_index("s")
    sA = (a0, a1, a2)
    sB = (b0, b1, b2)
    pltpu.sync_copy(P_hbm.at[pl.ds(s * (NPG // 16), NPG // 16)],
                    P_sh.at[pl.ds(s * (NPG // 16), NPG // 16)])
    plsc.subcore_barrier()

    def chunk_of(g, b):
        return w * CPW + g * NBUF + b

    def group(g, _):
        cds, css = [], []
        for b in range(NBUF):
            e0 = chunk_of(g, b) * CHUNK
            cds.append(pltpu.async_copy(dst1_hbm.at[pl.ds(e0, CHUNK)],
                                        idxd_v.at[b], sA[b]))
            css.append(pltpu.async_copy(src1_hbm.at[pl.ds(e0, CHUNK)],
                                        idxs_v.at[b], sB[b]))
        cps = []
        for b in range(NBUF):
            cds[b].wait()
            cps.append(pltpu.async_copy(P_sh.at[idxd_v.at[b]],
                                        rows_v.at[b], sA[b]))
        cqs = []
        for b in range(NBUF):
            css[b].wait()
            cps[b].wait()
            cqs.append(pltpu.async_copy(Q_hbm.at[idxs_v.at[b]],
                                        rows_v.at[b], sB[b], add=True))
        cws = []
        for b in range(NBUF):
            cqs[b].wait()
            cws.append(pltpu.async_copy(
                rows_v.at[b],
                t_hbm.at[pl.ds(chunk_of(g, b) * CHUNK, CHUNK)], sA[b]))
        for b in range(NBUF):
            cws[b].wait()
        return 0

    lax.fori_loop(0, NGRP, group, 0)

    e0 = (w * CPW + NGRP * NBUF) * CHUNK
    pltpu.sync_copy(dst1_hbm.at[pl.ds(e0, CHUNK)], idxd_v.at[0])
    pltpu.sync_copy(src1_hbm.at[pl.ds(e0, CHUNK)], idxs_v.at[0])
    pltpu.sync_copy(P_sh.at[idxd_v.at[0]], rows_v.at[0])
    pltpu.sync_copy(Q_hbm.at[idxs_v.at[0]], rows_v.at[0], add=True)
    pltpu.sync_copy(rows_v.at[0], t_hbm.at[pl.ds(e0, CHUNK)])


def _sc_gather(P, Q, dst1, src1):
    return _sc_gather_kernel(P, Q, dst1, src1)


@functools.partial(
    pl.kernel,
    out_type=jax.ShapeDtypeStruct((2, NP, H), jnp.float32),
    mesh=_MESH,
    scratch_types=[
        pltpu.VMEM((CPW, CHUNK), jnp.int32),
        pltpu.VMEM((NBUF_S, CHUNK, H), jnp.float32),
        pltpu.VMEM_SHARED((NP, H), jnp.float32),
        pltpu.SemaphoreType.DMA,
        pltpu.SemaphoreType.DMA,
    ],
)
def _sc_scatter_kernel(m2_hbm, dst2_hbm, z_hbm, agg2_hbm,
                       idx_v, buf_v, agg_sh, s0, s1):
    c = lax.axis_index("c")
    s = lax.axis_index("s")
    sems = (s0, s1)
    pltpu.sync_copy(z_hbm, agg_sh.at[pl.ds(s * NPS, NPS)])
    plsc.subcore_barrier()
    w = _wid()
    pltpu.sync_copy(dst2_hbm.at[pl.ds(w * CPW, CPW)], idx_v)

    def group(g, _):
        crs = [pltpu.async_copy(
                   m2_hbm.at[pl.ds((w * CPW + g * NBUF_S + b) * CHUNK, CHUNK)],
                   buf_v.at[b], sems[b])
               for b in range(NBUF_S)]
        css = []
        for b in range(NBUF_S):
            crs[b].wait()
            css.append(pltpu.async_copy(buf_v.at[b],
                                        agg_sh.at[idx_v.at[g * NBUF_S + b]],
                                        sems[b], add=True))
        for b in range(NBUF_S):
            css[b].wait()
        return 0

    lax.fori_loop(0, NGRP_S, group, 0)
    plsc.subcore_barrier()
    pltpu.sync_copy(agg_sh.at[pl.ds(s * NPS, NPS)],
                    agg2_hbm.at[c, pl.ds(s * NPS, NPS)])


def _sc_scatter(m2, dst2, z):
    return _sc_scatter_kernel(m2, dst2, z)


@functools.partial(
    pl.kernel,
    out_type=jax.ShapeDtypeStruct((2, NP, H), jnp.float32),
    mesh=_MESH,
    scratch_types=[
        pltpu.VMEM((CPW, CHUNK), jnp.int32),
        pltpu.VMEM((CHUNK, H), jnp.float32),
        pltpu.VMEM_SHARED((NP, H), jnp.float32),
        pltpu.SemaphoreType.DMA,
        pltpu.SemaphoreType.DMA,
        pltpu.SemaphoreType.DMA,
        pltpu.SemaphoreType.DMA,
    ],
)
def _sc_deg_kernel(dst2_hbm, ones_hbm, z16_hbm, deg2_hbm,
                   idx_v, ones_v, deg_sh, s0, s1, s2, s3):
    c = lax.axis_index("c")
    s = lax.axis_index("s")
    sems = (s0, s1, s2, s3)
    pltpu.sync_copy(z16_hbm, deg_sh.at[pl.ds(s * NPS, NPS)])
    pltpu.sync_copy(ones_hbm, ones_v)
    plsc.subcore_barrier()
    w = _wid()
    pltpu.sync_copy(dst2_hbm.at[pl.ds(w * CPW, CPW)], idx_v)

    def group(g, _):
        cs = [pltpu.async_copy(ones_v, deg_sh.at[idx_v.at[g * 4 + b]],
                               sems[b], add=True)
              for b in range(4)]
        for b in range(4):
            cs[b].wait()
        return 0

    lax.fori_loop(0, CPW // 4, group, 0)
    plsc.subcore_barrier()
    pltpu.sync_copy(deg_sh.at[pl.ds(s * NPS, NPS)],
                    deg2_hbm.at[c, pl.ds(s * NPS, NPS)])


def _sc_deg(dst2):
    ones = jnp.ones((CHUNK, H), jnp.float32)
    z16 = jnp.zeros((NPS, H), jnp.float32)
    return _sc_deg_kernel(dst2, ones, z16)


def kernel(x, pos, edge_index, batch, emb_W1, emb_b1, emb_W2, emb_b2,
           msg1_W, msg1_b, msg2_W, msg2_b, upd1_W, upd1_b, upd2_W, upd2_b,
           dbl_W, dbl_b, conv1_W, conv1_b, conv2_W, conv2_b):
    u = x
    pos_x = pos[:, 1:2] / PDE_L
    pos_t = pos[:, 0:1] / PDE_TMAX
    F = jnp.pad(jnp.concatenate([u, pos_x, pos_t], axis=1),
                ((0, NP - N), (0, 0)))
    src1 = jnp.pad(edge_index[0], (0, EP - E), constant_values=N)
    dst1 = jnp.pad(edge_index[1], (0, EP - E), constant_values=N)
    src2 = src1.reshape(NCHUNKS, CHUNK)
    dst2 = dst1.reshape(NCHUNKS, CHUNK)
    zN = jnp.zeros((NPS, H), jnp.float32)

    WPh = msg1_W[:, 0:H, :]
    WQh = msg1_W[:, H:2 * H, :]
    WPf = msg1_W[:, 2 * H:2 * H + 52, :]
    WQf = jnp.concatenate(
        [-msg1_W[:, 2 * H:2 * H + 51, :],
         jnp.zeros((NLAYERS, 1, H), jnp.float32)], axis=1)
    Uh = upd1_W[:, 0:H, :]
    Ua = upd1_W[:, H:2 * H, :]
    Uv = upd1_W[:, 2 * H:2 * H + 1, :]

    o_, i_, k_, j_ = np.meshgrid(np.arange(8), np.arange(2), np.arange(16),
                                 np.arange(38), indexing="ij")
    K1 = jnp.zeros((2 * H, 320), jnp.float32).at[
        (i_ * H + 3 * j_ + k_).ravel(), (o_ * 38 + j_).ravel()
    ].set(conv1_W[o_.ravel(), i_.ravel(), k_.ravel()])
    b1r = jnp.zeros((320,), jnp.float32).at[
        (o_ * 38 + j_).ravel()].set(conv1_b[o_.ravel()])
    c_, o2_, j2_, jj_ = np.meshgrid(np.arange(2), np.arange(8),
                                    np.arange(14), np.arange(25),
                                    indexing="ij")
    K2 = jnp.zeros((320, 64), jnp.float32).at[
        (o2_ * 38 + jj_ + j2_).ravel(), (c_ * 25 + jj_).ravel()
    ].add(conv2_W[c_.ravel(), o2_.ravel(), j2_.ravel()])
    b2r = jnp.zeros((64,), jnp.float32).at[
        (c_ * 25 + jj_).ravel()].set(conv2_b[c_.ravel()])
    dtv = jnp.zeros((64,), jnp.float32).at[np.arange(50)].set(
        np.tile(PDE_DT * (np.arange(25) + 1.0), 2).astype(np.float32))
    upad = jnp.pad(u, ((0, NP - N), (0, 64 - 2 * TW)))

    deg2 = _sc_deg(dst2)
    h, P, Q, degi = _enc_call(F, deg2, emb_W1, emb_b1, emb_W2, emb_b2,
                              WPh[0], WPf[0], WQh[0], WQf[0], msg1_b[0])
    for l in range(NLAYERS):
        t = _sc_gather(P, Q, dst1, src1)
        m2 = _msg_call(t, msg2_W[l], msg2_b[l])
        agg2 = _sc_scatter(m2, dst2, zN)
        if l < NLAYERS - 1:
            h, P, Q = _upd_call(h, F, agg2, degi,
                                Uh[l], Ua[l], Uv[l], upd1_b[l],
                                upd2_W[l], upd2_b[l],
                                WPh[l + 1], WPf[l + 1], WQh[l + 1],
                                WQf[l + 1], msg1_b[l + 1])
        else:
            outp = _fin_call(h, F, agg2, degi,
                             Uh[l], Ua[l], Uv[l], upd1_b[l],
                             upd2_W[l], upd2_b[l],
                             dbl_W, dbl_b, K1, b1r, K2, b2r, upad, dtv)
    return outp[:N, :2 * TW]
